# Initial kernel scaffold; baseline (speedup 1.0000x reference)
import jax
import jax.numpy as jnp
from jax.experimental import pallas as pl


def kernel(x, a, W, b, prelu_w):
    raise NotImplementedError("write your pallas kernel here")


# speedup vs baseline: 1.0290x; 1.0290x over previous
import jax
import jax.numpy as jnp
from jax.experimental import pallas as pl
from jax.experimental.pallas import tpu as pltpu


def _gcn_body(x_ref, wt_ref, b_ref, p_ref, a_ref, out_ref, xtheta_ref):
    @pl.when(pl.program_id(0) == 0)
    def _():
        xtheta_ref[...] = (
            jnp.dot(x_ref[...], wt_ref[...], preferred_element_type=jnp.float32)
            + b_ref[...]
        )

    acc = jnp.dot(a_ref[...], xtheta_ref[...], preferred_element_type=jnp.float32)
    p = p_ref[0, 0]
    out_ref[...] = jnp.where(acc >= 0, acc, p * acc)


def kernel(x, a, W, b, prelu_w):
    n, d_in = x.shape[1], x.shape[2]
    d_out = W.shape[0]
    x2 = x[0]
    wt = W.T
    b2 = b.reshape(1, d_out)
    p2 = prelu_w.reshape(1, 1)

    bm = 400
    grid = n // bm

    out = pl.pallas_call(
        _gcn_body,
        grid=(grid,),
        in_specs=[
            pl.BlockSpec((n, d_in), lambda i: (0, 0)),
            pl.BlockSpec((d_in, d_out), lambda i: (0, 0)),
            pl.BlockSpec((1, d_out), lambda i: (0, 0)),
            pl.BlockSpec((1, 1), lambda i: (0, 0)),
            pl.BlockSpec((bm, n), lambda i: (i, 0)),
        ],
        out_specs=pl.BlockSpec((bm, d_out), lambda i: (i, 0)),
        out_shape=jax.ShapeDtypeStruct((n, d_out), jnp.float32),
        scratch_shapes=[pltpu.VMEM((n, d_out), jnp.float32)],
    )(x2, wt, b2, p2, a)
    return out[None]
